# baseline (speedup 1.0000x reference)
import functools

import jax
import jax.numpy as jnp
from jax import lax
from jax.experimental import pallas as pl
from jax.experimental.pallas import tpu as pltpu
from jax.experimental.pallas import tpu_sc as plsc

_B, _N, _M = 8, 2048, 2048
_SCALE = 1.0 / (_B * _N)


_BSC = 2
_PAD = 16
_NP = _N + _PAD
_NC, _NS = 2, 16
_NW = _NC * _NS
_CPB = _NS
_CHUNK = _N // _CPB
_GB = 4
_NGB = _CHUNK // (_GB * 16)
_QT = _N // _CPB
_INF = 3.4e38


@functools.partial(
    pl.kernel,
    out_type=jax.ShapeDtypeStruct((_NW * 16,), jnp.float32),
    mesh=plsc.VectorSubcoreMesh(core_axis_name="c", subcore_axis_name="s"),
    scratch_types=[
        pltpu.VMEM((_CHUNK,), jnp.float32),
        pltpu.VMEM((_CHUNK,), jnp.float32),
        pltpu.VMEM((_CHUNK,), jnp.float32),
        pltpu.VMEM((_NP,), jnp.float32),
        pltpu.VMEM((_NP,), jnp.float32),
        pltpu.VMEM((_NP,), jnp.float32),
        pltpu.VMEM((_NP,), jnp.float32),
        pltpu.VMEM((_CPB * _QT,), jnp.float32),
        pltpu.VMEM((16,), jnp.float32),
        pltpu.VMEM_SHARED((_NS * _N,), jnp.float32),
    ],
)
def _sc_chamfer(flat_h, out_hbm,
                cx, cy, cz, fx, fy, fz, d2min, comb, sum_v, shared):
    c = lax.axis_index("c")
    s = lax.axis_index("s")
    b = c
    ck = s
    wid = c * _NS + s

    for d, dst in enumerate((cx, cy, cz)):
        off = (d * _BSC + b) * _NP + ck * _CHUNK
        pltpu.sync_copy(flat_h.at[pl.ds(off, _CHUNK)], dst)
    for d, dst in enumerate((fx, fy, fz)):
        off = ((3 + d) * _BSC + b) * _NP
        pltpu.sync_copy(flat_h.at[pl.ds(off, _NP)], dst)

    inf_v = jnp.full((16,), _INF, jnp.float32)

    def initbody(i, carry):
        d2min[pl.ds(i * 16, 16)] = inf_v
        return carry

    lax.fori_loop(0, _NP // 16, initbody, 0)

    def gbody(g, total1):
        base = g * _GB * 16
        px = [cx[pl.ds(base + k * 16, 16)] for k in range(_GB)]
        py = [cy[pl.ds(base + k * 16, 16)] for k in range(_GB)]
        pz = [cz[pl.ds(base + k * 16, 16)] for k in range(_GB)]
        init = tuple(jnp.full((16,), _INF, jnp.float32) for k in range(_GB))

        def obody(o, mins):
            tx = fx[pl.ds(o, 16)]
            ty = fy[pl.ds(o, 16)]
            tz = fz[pl.ds(o, 16)]
            new = []
            d2s = []
            for k in range(_GB):
                dx = px[k] - tx
                dy = py[k] - ty
                dz = pz[k] - tz
                d2 = dx * dx + dy * dy + dz * dz
                d2s.append(d2)
                new.append(jnp.minimum(mins[k], d2))
            colmin = jnp.minimum(jnp.minimum(d2min[pl.ds(o, 16)],
                                             jnp.minimum(d2s[0], d2s[1])),
                                 jnp.minimum(d2s[2], d2s[3]))
            d2min[pl.ds(o, 16)] = colmin
            return tuple(new)

        mins = lax.fori_loop(0, _N, obody, init, unroll=4)
        for k in range(_GB):
            total1 = total1 + mins[k]
        return total1

    total1 = lax.fori_loop(0, _NGB, gbody, jnp.zeros((16,), jnp.float32))

    d2min[pl.ds(0, 16)] = jnp.minimum(d2min[pl.ds(0, 16)],
                                      d2min[pl.ds(_N, 16)])

    pltpu.sync_copy(d2min.at[pl.ds(0, _N)], shared.at[pl.ds(s * _N, _N)])
    plsc.subcore_barrier()
    for r in range(_CPB):
        pltpu.sync_copy(shared.at[pl.ds(r * _N + s * _QT, _QT)],
                        comb.at[pl.ds(r * _QT, _QT)])

    def combbody(i, total2):
        m = comb[pl.ds(i * 16, 16)]
        for r in range(1, _CPB):
            m = jnp.minimum(m, comb[pl.ds(r * _QT + i * 16, 16)])
        return total2 + m

    total2 = lax.fori_loop(0, _QT // 16, combbody,
                           jnp.zeros((16,), jnp.float32))

    sum_v[...] = (total1 + total2) * _SCALE
    pltpu.sync_copy(sum_v, out_hbm.at[pl.ds(wid * 16, 16)])


_BTC = _B - _BSC


def _tc_body(p_ref, tT_ref, out_ref):
    b = pl.program_id(0)
    p = p_ref[0]
    tT = tT_ref[0]
    d2 = (p[:, 0:1] - tT[0:1, :]) ** 2
    d2 += (p[:, 1:2] - tT[1:2, :]) ** 2
    d2 += (p[:, 2:3] - tT[2:3, :]) ** 2
    s1 = jnp.sum(jnp.min(d2, axis=1))
    s2 = jnp.sum(jnp.min(d2, axis=0))

    @pl.when(b == 0)
    def _():
        out_ref[0, 0] = 0.0

    out_ref[0, 0] += (s1 + s2) * _SCALE


def _tc_chamfer(pred, tT):
    out = pl.pallas_call(
        _tc_body,
        grid=(_BTC,),
        in_specs=[
            pl.BlockSpec((1, _N, 3), lambda b: (b + _BSC, 0, 0)),
            pl.BlockSpec((1, 3, _M), lambda b: (b + _BSC, 0, 0)),
        ],
        out_specs=pl.BlockSpec(memory_space=pltpu.SMEM),
        out_shape=jax.ShapeDtypeStruct((1, 1), jnp.float32),
        compiler_params=pltpu.CompilerParams(
            dimension_semantics=("arbitrary",),
        ),
    )(pred, tT)
    return out[0, 0]


def kernel(pred, target):
    pred = pred.astype(jnp.float32)
    target = target.astype(jnp.float32)
    pt = jnp.stack([pred[:_BSC], target[:_BSC]])
    ptp = jnp.concatenate([pt, pt[:, :, :_PAD]], axis=2)
    flat = ptp.transpose(0, 3, 1, 2).reshape(-1)
    sc_parts = _sc_chamfer(flat)
    tc_part = _tc_chamfer(pred, target.swapaxes(1, 2))
    return jnp.sum(sc_parts) + tc_part


# speedup vs baseline: 1.2746x; 1.2746x over previous
import functools

import jax
import jax.numpy as jnp
from jax import lax
from jax.experimental import pallas as pl
from jax.experimental.pallas import tpu as pltpu
from jax.experimental.pallas import tpu_sc as plsc

_B, _N, _M = 8, 2048, 2048
_SCALE = 1.0 / (_B * _N)


_BSC = 2
_PAD = 16
_NP = _N + _PAD
_NC, _NS = 2, 16
_NW = _NC * _NS
_CPB = _NS
_CHUNK = _N // _CPB
_GB = 8
_QT = _N // _CPB
_INF = 3.4e38


@functools.partial(
    pl.kernel,
    out_type=jax.ShapeDtypeStruct((_NW * 16,), jnp.float32),
    mesh=plsc.VectorSubcoreMesh(core_axis_name="c", subcore_axis_name="s"),
    scratch_types=[
        pltpu.VMEM((_CHUNK,), jnp.float32),
        pltpu.VMEM((_CHUNK,), jnp.float32),
        pltpu.VMEM((_CHUNK,), jnp.float32),
        pltpu.VMEM((_NP,), jnp.float32),
        pltpu.VMEM((_NP,), jnp.float32),
        pltpu.VMEM((_NP,), jnp.float32),
        pltpu.VMEM((_NP,), jnp.float32),
        pltpu.VMEM((_CPB * _QT,), jnp.float32),
        pltpu.VMEM((16,), jnp.float32),
        pltpu.VMEM_SHARED((_NS * _N,), jnp.float32),
    ],
)
def _sc_chamfer(flat_h, out_hbm,
                cx, cy, cz, fx, fy, fz, d2min, comb, sum_v, shared):
    c = lax.axis_index("c")
    s = lax.axis_index("s")
    b = c
    ck = s
    wid = c * _NS + s

    for d, dst in enumerate((cx, cy, cz)):
        off = (d * _BSC + b) * _NP + ck * _CHUNK
        pltpu.sync_copy(flat_h.at[pl.ds(off, _CHUNK)], dst)
    for d, dst in enumerate((fx, fy, fz)):
        off = ((3 + d) * _BSC + b) * _NP
        pltpu.sync_copy(flat_h.at[pl.ds(off, _NP)], dst)

    inf_v = jnp.full((16,), _INF, jnp.float32)

    def initbody(i, carry):
        d2min[pl.ds(i * 16, 16)] = inf_v
        return carry

    lax.fori_loop(0, _NP // 16, initbody, 0)

    px = [cx[pl.ds(k * 16, 16)] for k in range(_GB)]
    py = [cy[pl.ds(k * 16, 16)] for k in range(_GB)]
    pz = [cz[pl.ds(k * 16, 16)] for k in range(_GB)]
    init = tuple(jnp.full((16,), _INF, jnp.float32) for k in range(_GB))

    def obody(o, mins):
        tx = fx[pl.ds(o, 16)]
        ty = fy[pl.ds(o, 16)]
        tz = fz[pl.ds(o, 16)]
        new = []
        d2s = []
        for k in range(_GB):
            dx = px[k] - tx
            dy = py[k] - ty
            dz = pz[k] - tz
            d2 = dx * dx + dy * dy + dz * dz
            d2s.append(d2)
            new.append(jnp.minimum(mins[k], d2))
        while len(d2s) > 1:
            d2s = [jnp.minimum(a, b) for a, b in zip(d2s[::2], d2s[1::2])]
        d2min[pl.ds(o, 16)] = jnp.minimum(d2min[pl.ds(o, 16)], d2s[0])
        return tuple(new)

    mins = lax.fori_loop(0, _N, obody, init)
    total1 = jnp.zeros((16,), jnp.float32)
    for k in range(_GB):
        total1 = total1 + mins[k]

    d2min[pl.ds(0, 16)] = jnp.minimum(d2min[pl.ds(0, 16)],
                                      d2min[pl.ds(_N, 16)])

    pltpu.sync_copy(d2min.at[pl.ds(0, _N)], shared.at[pl.ds(s * _N, _N)])
    plsc.subcore_barrier()
    for r in range(_CPB):
        pltpu.sync_copy(shared.at[pl.ds(r * _N + s * _QT, _QT)],
                        comb.at[pl.ds(r * _QT, _QT)])

    def combbody(i, total2):
        m = comb[pl.ds(i * 16, 16)]
        for r in range(1, _CPB):
            m = jnp.minimum(m, comb[pl.ds(r * _QT + i * 16, 16)])
        return total2 + m

    total2 = lax.fori_loop(0, _QT // 16, combbody,
                           jnp.zeros((16,), jnp.float32))

    sum_v[...] = (total1 + total2) * _SCALE
    pltpu.sync_copy(sum_v, out_hbm.at[pl.ds(wid * 16, 16)])


_BTC = _B - _BSC


def _tc_body(p_ref, tT_ref, out_ref):
    b = pl.program_id(0)
    p = p_ref[0]
    tT = tT_ref[0]
    d2 = (p[:, 0:1] - tT[0:1, :]) ** 2
    d2 += (p[:, 1:2] - tT[1:2, :]) ** 2
    d2 += (p[:, 2:3] - tT[2:3, :]) ** 2
    s1 = jnp.sum(jnp.min(d2, axis=1))
    s2 = jnp.sum(jnp.min(d2, axis=0))

    @pl.when(b == 0)
    def _():
        out_ref[0, 0] = 0.0

    out_ref[0, 0] += (s1 + s2) * _SCALE


def _tc_chamfer(pred, tT):
    out = pl.pallas_call(
        _tc_body,
        grid=(_BTC,),
        in_specs=[
            pl.BlockSpec((1, _N, 3), lambda b: (b + _BSC, 0, 0)),
            pl.BlockSpec((1, 3, _M), lambda b: (b, 0, 0)),
        ],
        out_specs=pl.BlockSpec(memory_space=pltpu.SMEM),
        out_shape=jax.ShapeDtypeStruct((1, 1), jnp.float32),
        compiler_params=pltpu.CompilerParams(
            dimension_semantics=("arbitrary",),
        ),
    )(pred, tT)
    return out[0, 0]


def kernel(pred, target):
    pred = pred.astype(jnp.float32)
    target = target.astype(jnp.float32)
    pt = jnp.stack([pred[:_BSC], target[:_BSC]])
    ptp = jnp.concatenate([pt, pt[:, :, :_PAD]], axis=2)
    flat = ptp.transpose(0, 3, 1, 2).reshape(-1)
    sc_parts = _sc_chamfer(flat)
    tc_part = _tc_chamfer(pred, target[_BSC:].swapaxes(1, 2))
    return jnp.sum(sc_parts) + tc_part


# speedup vs baseline: 1.3485x; 1.0580x over previous
import jax
import jax.numpy as jnp
from jax.experimental import pallas as pl
from jax.experimental.pallas import tpu as pltpu

_B, _N, _M = 8, 2048, 2048


def _chamfer_body(p_ref, t_ref, out_ref):
    b = pl.program_id(0)
    p = p_ref[0]
    tT = t_ref[0].T
    d2 = (p[:, 0:1] - tT[0:1, :]) ** 2
    d2 += (p[:, 1:2] - tT[1:2, :]) ** 2
    d2 += (p[:, 2:3] - tT[2:3, :]) ** 2
    s1 = jnp.sum(jnp.min(d2, axis=1))
    s2 = jnp.sum(jnp.min(d2, axis=0))

    @pl.when(b == 0)
    def _():
        out_ref[0, 0] = 0.0

    out_ref[0, 0] += (s1 + s2) * (1.0 / (_B * _N))


def kernel(pred, target):
    pred = pred.astype(jnp.float32)
    target = target.astype(jnp.float32)
    out = pl.pallas_call(
        _chamfer_body,
        grid=(_B,),
        in_specs=[
            pl.BlockSpec((1, _N, 3), lambda b: (b, 0, 0)),
            pl.BlockSpec((1, _M, 3), lambda b: (b, 0, 0)),
        ],
        out_specs=pl.BlockSpec(memory_space=pltpu.SMEM),
        out_shape=jax.ShapeDtypeStruct((1, 1), jnp.float32),
        compiler_params=pltpu.CompilerParams(
            dimension_semantics=("arbitrary",),
        ),
    )(pred, target)
    return out[0, 0]


# speedup vs baseline: 1.4501x; 1.0753x over previous
import jax
import jax.numpy as jnp
from jax.experimental import pallas as pl
from jax.experimental.pallas import tpu as pltpu

_B, _N, _M = 8, 2048, 2048


def _chamfer_body(p_ref, tT_ref, out_ref):
    b = pl.program_id(0)
    p = p_ref[0]
    tT = tT_ref[0]
    d2 = (p[:, 0:1] - tT[0:1, :]) ** 2
    d2 += (p[:, 1:2] - tT[1:2, :]) ** 2
    d2 += (p[:, 2:3] - tT[2:3, :]) ** 2
    s1 = jnp.sum(jnp.min(d2, axis=1))
    s2 = jnp.sum(jnp.min(d2, axis=0))

    @pl.when(b == 0)
    def _():
        out_ref[0, 0] = 0.0

    out_ref[0, 0] += (s1 + s2) * (1.0 / (_B * _N))


def kernel(pred, target):
    pred = pred.astype(jnp.float32)
    tT = target.astype(jnp.float32).swapaxes(1, 2)
    out = pl.pallas_call(
        _chamfer_body,
        grid=(_B,),
        in_specs=[
            pl.BlockSpec((1, _N, 3), lambda b: (b, 0, 0)),
            pl.BlockSpec((1, 3, _M), lambda b: (b, 0, 0)),
        ],
        out_specs=pl.BlockSpec(memory_space=pltpu.SMEM),
        out_shape=jax.ShapeDtypeStruct((1, 1), jnp.float32),
        compiler_params=pltpu.CompilerParams(
            dimension_semantics=("arbitrary",),
        ),
    )(pred, tT)
    return out[0, 0]
